# Initial kernel scaffold; baseline (speedup 1.0000x reference)
import jax
import jax.numpy as jnp
from jax.experimental import pallas as pl


def kernel(x, fc_weight, bias):
    raise NotImplementedError("write your pallas kernel here")


# speedup vs baseline: 1.5798x; 1.5798x over previous
import functools

import jax
import jax.numpy as jnp
from jax import lax
from jax.experimental import pallas as pl
from jax.experimental.pallas import tpu as pltpu
from jax.experimental.pallas import tpu_sc as plsc

NUM_FIELDS = 26
FIELD_DIM = 40000
BATCH = 16384
NC = 2
NS = 16
B_PER_CORE = BATCH // NC
B_PER_TILE = B_PER_CORE // NS
L = 16


def _sc_body(xT, table, bias, out, tab1_v, tab2_v, idx1_v, idx2_v,
             part_v, red_v, out_v, bias_v, shared):
    c = lax.axis_index("c")
    s = lax.axis_index("s")
    base_b = c * B_PER_CORE

    f1 = s
    f2 = s + NS

    pltpu.sync_copy(bias, bias_v)
    pltpu.sync_copy(xT.at[f1, pl.ds(base_b, B_PER_CORE)], idx1_v)
    pltpu.sync_copy(table.at[pl.ds(f1 * FIELD_DIM, FIELD_DIM)], tab1_v)

    @pl.loop(0, B_PER_CORE // L)
    def _gather1(j):
        sl = pl.ds(j * L, L)
        part_v[sl] = plsc.load_gather(tab1_v, [idx1_v[sl]])

    @pl.when(s < NUM_FIELDS - NS)
    def _second_field():
        pltpu.sync_copy(xT.at[f2, pl.ds(base_b, B_PER_CORE)], idx2_v)
        pltpu.sync_copy(table.at[pl.ds(f2 * FIELD_DIM, FIELD_DIM)], tab2_v)

        @pl.loop(0, B_PER_CORE // L)
        def _gather2(j):
            sl = pl.ds(j * L, L)
            part_v[sl] = part_v[sl] + plsc.load_gather(tab2_v, [idx2_v[sl]])

    pltpu.sync_copy(part_v, shared.at[s])
    plsc.subcore_barrier()

    for t in range(NS):
        pltpu.sync_copy(shared.at[t, pl.ds(s * B_PER_TILE, B_PER_TILE)],
                        red_v.at[t])

    bias_vec = bias_v[...]
    for j in range(B_PER_TILE // L):
        sl = pl.ds(j * L, L)
        acc = red_v[0, sl]
        for t in range(1, NS):
            acc = acc + red_v[t, sl]
        out_v[sl] = acc + bias_vec

    pltpu.sync_copy(out_v, out.at[pl.ds(base_b + s * B_PER_TILE, B_PER_TILE)])


_sc_kernel = functools.partial(
    pl.kernel,
    out_type=jax.ShapeDtypeStruct((BATCH,), jnp.float32),
    mesh=plsc.VectorSubcoreMesh(core_axis_name="c", subcore_axis_name="s",
                                num_cores=NC, num_subcores=NS),
    scratch_types=[
        pltpu.VMEM((FIELD_DIM,), jnp.float32),
        pltpu.VMEM((FIELD_DIM,), jnp.float32),
        pltpu.VMEM((B_PER_CORE,), jnp.int32),
        pltpu.VMEM((B_PER_CORE,), jnp.int32),
        pltpu.VMEM((B_PER_CORE,), jnp.float32),
        pltpu.VMEM((NS, B_PER_TILE), jnp.float32),
        pltpu.VMEM((B_PER_TILE,), jnp.float32),
        pltpu.VMEM((L,), jnp.float32),
        pltpu.VMEM_SHARED((NS, B_PER_CORE), jnp.float32),
    ],
    compiler_params=pltpu.CompilerParams(needs_layout_passes=False),
)(_sc_body)


@jax.jit
def kernel(x, fc_weight, bias):
    xT = x.astype(jnp.int32).T
    table = fc_weight.reshape(-1).astype(jnp.float32)
    bias16 = jnp.broadcast_to(bias.astype(jnp.float32), (L,))
    out = _sc_kernel(xT, table, bias16)
    return out.reshape(BATCH, 1)
